# baseline (speedup 1.0000x reference)
import functools

import jax
import jax.numpy as jnp
from jax import lax
from jax.experimental import pallas as pl
from jax.experimental.pallas import tpu as pltpu
from jax.experimental.pallas import tpu_sc as plsc

_VOCAB = 1000
_VPAD = 1024
_LO = 896
_HI = _VPAD - _LO
_EMB = 8
_BATCH = 1024
_SEQ = 50
_SEQ_PAD = 56
_MAIN = 48
_REM = 8

_NC = 2
_NS = 16
_NW = _NC * _NS
_BATCH_PER_W = _BATCH // _NW
_IDS_PER_W = _BATCH_PER_W * _SEQ_PAD


def _table_body(embed_ref, w_ref, b_ref, lo_ref, hi_ref):
    t = lax.dot_general(
        embed_ref[...], w_ref[...],
        (((1,), (1,)), ((), ())),
        preferred_element_type=jnp.float32,
    ) + b_ref[...]
    lo_ref[...] = t[:, :_LO]
    hi_ref[...] = t[:, _LO:]


def _make_tables(embed, w_pad, b_pad):
    return pl.pallas_call(
        _table_body,
        out_shape=(
            jax.ShapeDtypeStruct((_VOCAB, _LO), jnp.float32),
            jax.ShapeDtypeStruct((_VOCAB, _HI), jnp.float32),
        ),
    )(embed, w_pad, b_pad.reshape(1, _VPAD))


_sc_mesh = plsc.VectorSubcoreMesh(core_axis_name="c", subcore_axis_name="s")

_SC_SCRATCH = [
    pltpu.VMEM((_IDS_PER_W,), jnp.int32),
    pltpu.VMEM((_SEQ, _VOCAB), jnp.float32),
    pltpu.VMEM((_SEQ, _VOCAB), jnp.float32),
    pltpu.VMEM((_REM, _LO), jnp.float32),
    pltpu.VMEM((_SEQ, _HI), jnp.float32),
    pltpu.SemaphoreType.DMA,
    pltpu.SemaphoreType.DMA,
    pltpu.SemaphoreType.DMA,
    pltpu.SemaphoreType.DMA,
]


def _sc_gather_body(lo_hbm, hi_hbm, ids_hbm, out_hbm,
                    idx_v, buf0, buf1, bufr, bhi, sl0, sl1, sr, sh):
    wid = lax.axis_index("s") * _NC + lax.axis_index("c")
    batch0 = wid * _BATCH_PER_W
    pltpu.sync_copy(ids_hbm.at[pl.ds(wid * _IDS_PER_W, _IDS_PER_W)], idx_v)

    def idx_at(g, local_off, n):
        off = pl.multiple_of(g * _SEQ_PAD + local_off, 8)
        return idx_v.at[pl.ds(off, n)]

    def lo_dst(buf):
        return buf.at[pl.ds(0, _MAIN), pl.ds(0, _LO)]

    def start_lo(g, buf, sem):
        pltpu.async_copy(lo_hbm.at[idx_at(g, 0, _MAIN)], lo_dst(buf), sem)

    def wait_lo(buf, sem):
        pltpu.make_async_copy(
            lo_hbm.at[idx_at(0, 0, _MAIN)], lo_dst(buf), sem
        ).wait()

    def start_remhi(g):
        pltpu.async_copy(lo_hbm.at[idx_at(g, _MAIN, _REM)], bufr, sr)
        pltpu.async_copy(hi_hbm.at[idx_at(g, 0, _SEQ)], bhi, sh)

    def wait_remhi():
        pltpu.make_async_copy(lo_hbm.at[idx_at(0, _MAIN, _REM)], bufr, sr).wait()
        pltpu.make_async_copy(hi_hbm.at[idx_at(0, 0, _SEQ)], bhi, sh).wait()

    def fix(buf):
        def rep_row(j, carry):
            for k in range(_LO // 16):
                buf[_MAIN + j, pl.ds(16 * k, 16)] = bufr[j, pl.ds(16 * k, 16)]
            return carry

        lax.fori_loop(0, _SEQ - _MAIN, rep_row, 0)

        def fix_row(r, carry):
            buf[r, pl.ds(_VOCAB - 16, 16)] = bhi[r, pl.ds(_VOCAB - 16 - _LO, 16)]
            for k in range(6):
                buf[r, pl.ds(_LO + 16 * k, 16)] = bhi[r, pl.ds(16 * k, 16)]
            return carry

        lax.fori_loop(0, _SEQ, fix_row, 0)

    def halfstep(g, buf, sem, other_buf, other_sem):
        @pl.when(g + 1 < _BATCH_PER_W)
        def _():
            start_lo(g + 1, other_buf, other_sem)

        wait_lo(buf, sem)
        wait_remhi()
        fix(buf)

        @pl.when(g + 1 < _BATCH_PER_W)
        def _():
            start_remhi(g + 1)

        pltpu.sync_copy(buf, out_hbm.at[batch0 + g])

    start_lo(0, buf0, sl0)
    start_remhi(0)

    def body(i, carry):
        g = 2 * i
        halfstep(g, buf0, sl0, buf1, sl1)
        halfstep(g + 1, buf1, sl1, buf0, sl0)
        return carry

    lax.fori_loop(0, _BATCH_PER_W // 2, body, 0)


_sc_gather = pl.kernel(
    _sc_gather_body,
    out_type=jax.ShapeDtypeStruct((_BATCH, _SEQ, _VOCAB), jnp.float32),
    mesh=_sc_mesh,
    scratch_types=_SC_SCRATCH,
)


def kernel(input_ids, embed, W, b):
    w_pad = jnp.pad(W, ((0, _VPAD - _VOCAB), (0, 0)))
    b_pad = jnp.pad(b, (0, _VPAD - _VOCAB))
    t_lo, t_hi = _make_tables(embed, w_pad, b_pad)
    ids_pad = jnp.pad(
        input_ids.astype(jnp.int32), ((0, 0), (0, _SEQ_PAD - _SEQ))
    ).reshape(_BATCH * _SEQ_PAD)
    return _sc_gather(t_lo, t_hi, ids_pad)


# speedup vs baseline: 1.2024x; 1.2024x over previous
import functools

import jax
import jax.numpy as jnp
from jax import lax
from jax.experimental import pallas as pl
from jax.experimental.pallas import tpu as pltpu
from jax.experimental.pallas import tpu_sc as plsc

_VOCAB = 1000
_VPAD = 1024
_EMB = 8
_BATCH = 1024
_SEQ = 50
_SEQ_PAD = 56
_HALF_A = 24
_HALF_B = _SEQ - _HALF_A

_NC = 2
_NS = 16
_NW = _NC * _NS
_BATCH_PER_W = _BATCH // _NW
_IDS_PER_W = _BATCH_PER_W * _SEQ_PAD


def _table_body(embed_ref, w_ref, b_ref, out_ref):
    t = lax.dot_general(
        embed_ref[...], w_ref[...],
        (((1,), (1,)), ((), ())),
        preferred_element_type=jnp.float32,
    ) + b_ref[...]
    out_ref[...] = t.reshape(_VOCAB, 8, 128)


def _make_table3(embed, w_pad, b_pad):
    return pl.pallas_call(
        _table_body,
        out_shape=jax.ShapeDtypeStruct((_VOCAB, 8, 128), jnp.float32),
    )(embed, w_pad, b_pad.reshape(1, _VPAD))


_sc_mesh = plsc.VectorSubcoreMesh(core_axis_name="c", subcore_axis_name="s")

_SC_SCRATCH = [
    pltpu.VMEM((_IDS_PER_W,), jnp.int32),
    pltpu.VMEM((_HALF_A, 8, 128), jnp.float32),
    pltpu.VMEM((_HALF_B, 8, 128), jnp.float32),
    pltpu.VMEM((_SEQ, _VOCAB), jnp.float32),
    pltpu.SemaphoreType.DMA,
    pltpu.SemaphoreType.DMA,
    pltpu.SemaphoreType.DMA,
]


def _sc_gather_body(t3_hbm, ids_hbm, out_hbm,
                    idx_v, b3a, b3b, buf, sa, sb, sw):
    wid = lax.axis_index("s") * _NC + lax.axis_index("c")
    batch0 = wid * _BATCH_PER_W
    pltpu.sync_copy(ids_hbm.at[pl.ds(wid * _IDS_PER_W, _IDS_PER_W)], idx_v)

    def idx_at(g, local_off, n):
        off = pl.multiple_of(g * _SEQ_PAD + local_off, 8)
        return idx_v.at[pl.ds(off, n)]

    def start(g):
        pltpu.async_copy(t3_hbm.at[idx_at(g, 0, _HALF_A)], b3a, sa)
        pltpu.async_copy(t3_hbm.at[idx_at(g, _HALF_A, _HALF_B)], b3b, sb)

    def wait_half(b3, sem):
        pltpu.make_async_copy(t3_hbm.at[idx_at(0, 0, b3.shape[0])], b3, sem).wait()

    def vec_half(b3, row0, nrows):
        def move_row(rr, carry):
            r = row0 + rr
            for i in range(7):
                for j in range(8):
                    buf[r, pl.ds(128 * i + 16 * j, 16)] = b3[rr, i, pl.ds(16 * j, 16)]
            buf[r, pl.ds(_VOCAB - 16, 16)] = b3[rr, 7, pl.ds(88, 16)]
            for j in range(6):
                buf[r, pl.ds(896 + 16 * j, 16)] = b3[rr, 7, pl.ds(16 * j, 16)]
            return carry

        lax.fori_loop(0, nrows, move_row, 0)

    def wait_write():
        pltpu.make_async_copy(buf, out_hbm.at[batch0], sw).wait()

    start(0)

    def body(g, carry):
        wait_half(b3a, sa)

        @pl.when(g > 0)
        def _():
            wait_write()

        vec_half(b3a, 0, _HALF_A)
        wait_half(b3b, sb)
        vec_half(b3b, _HALF_A, _HALF_B)
        pltpu.async_copy(buf, out_hbm.at[batch0 + g], sw)

        @pl.when(g + 1 < _BATCH_PER_W)
        def _():
            start(g + 1)

        return carry

    lax.fori_loop(0, _BATCH_PER_W, body, 0)
    wait_write()


_sc_gather = pl.kernel(
    _sc_gather_body,
    out_type=jax.ShapeDtypeStruct((_BATCH, _SEQ, _VOCAB), jnp.float32),
    mesh=_sc_mesh,
    scratch_types=_SC_SCRATCH,
)


def kernel(input_ids, embed, W, b):
    w_pad = jnp.pad(W, ((0, _VPAD - _VOCAB), (0, 0)))
    b_pad = jnp.pad(b, (0, _VPAD - _VOCAB))
    t3 = _make_table3(embed, w_pad, b_pad)
    ids_pad = jnp.pad(
        input_ids.astype(jnp.int32), ((0, 0), (0, _SEQ_PAD - _SEQ))
    ).reshape(_BATCH * _SEQ_PAD)
    return _sc_gather(t3, ids_pad)


# speedup vs baseline: 1.9073x; 1.5862x over previous
import functools

import jax
import jax.numpy as jnp
from jax import lax
from jax.experimental import pallas as pl
from jax.experimental.pallas import tpu as pltpu
from jax.experimental.pallas import tpu_sc as plsc

_VOCAB = 1000
_VPAD = 1024
_EMB = 8
_BATCH = 1024
_SEQ = 50
_SEQ_PAD = 56
_HALF_A = 24
_HALF_B = _SEQ - _HALF_A

_NC = 2
_NS = 16
_NW = _NC * _NS
_BATCH_PER_W = _BATCH // _NW
_IDS_PER_W = _BATCH_PER_W * _SEQ_PAD


def _table_body(embed_ref, w_ref, b_ref, out_ref):
    t = lax.dot_general(
        embed_ref[...], w_ref[...],
        (((1,), (1,)), ((), ())),
        preferred_element_type=jnp.float32,
    ) + b_ref[...]
    out_ref[...] = t.reshape(_VOCAB, 8, 128)


def _make_table3(embed, w_pad, b_pad):
    return pl.pallas_call(
        _table_body,
        out_shape=jax.ShapeDtypeStruct((_VOCAB, 8, 128), jnp.float32),
    )(embed, w_pad, b_pad.reshape(1, _VPAD))


_sc_mesh = plsc.VectorSubcoreMesh(core_axis_name="c", subcore_axis_name="s")

_SC_SCRATCH = [
    pltpu.VMEM((_IDS_PER_W,), jnp.int32),
    pltpu.VMEM((_HALF_A, 8, 128), jnp.float32),
    pltpu.VMEM((_HALF_B, 8, 128), jnp.float32),
    pltpu.VMEM((_SEQ, _VOCAB), jnp.float32),
    pltpu.SemaphoreType.DMA,
    pltpu.SemaphoreType.DMA,
    pltpu.SemaphoreType.DMA,
]


def _sc_gather_body(t3_hbm, ids_hbm, out_hbm,
                    idx_v, b3a, b3b, buf, sa, sb, sw):
    wid = lax.axis_index("s") * _NC + lax.axis_index("c")
    batch0 = wid * _BATCH_PER_W
    pltpu.sync_copy(ids_hbm.at[pl.ds(wid * _IDS_PER_W, _IDS_PER_W)], idx_v)

    def idx_at(g, local_off, n):
        off = pl.multiple_of(g * _SEQ_PAD + local_off, 8)
        return idx_v.at[pl.ds(off, n)]

    def start(g):
        pltpu.async_copy(t3_hbm.at[idx_at(g, 0, _HALF_A)], b3a, sa)
        pltpu.async_copy(t3_hbm.at[idx_at(g, _HALF_A, _HALF_B)], b3b, sb)

    def wait_half(b3, sem):
        pltpu.make_async_copy(t3_hbm.at[idx_at(0, 0, b3.shape[0])], b3, sem).wait()

    def vec_half(b3, row0, nrows):
        @plsc.parallel_loop(0, nrows, 1, unroll=2)
        def _(rr):
            r = row0 + rr
            for i in range(7):
                vals = [b3[rr, i, pl.ds(16 * j, 16)] for j in range(8)]
                for j in range(8):
                    buf[r, pl.ds(128 * i + 16 * j, 16)] = vals[j]
            vtail = b3[rr, 7, pl.ds(88, 16)]
            vals = [b3[rr, 7, pl.ds(16 * j, 16)] for j in range(6)]
            buf[r, pl.ds(_VOCAB - 16, 16)] = vtail
            for j in range(6):
                buf[r, pl.ds(896 + 16 * j, 16)] = vals[j]

    def wait_write():
        pltpu.make_async_copy(buf, out_hbm.at[batch0], sw).wait()

    start(0)

    def body(g, carry):
        wait_half(b3a, sa)

        @pl.when(g > 0)
        def _():
            wait_write()

        vec_half(b3a, 0, _HALF_A)
        wait_half(b3b, sb)
        vec_half(b3b, _HALF_A, _HALF_B)
        pltpu.async_copy(buf, out_hbm.at[batch0 + g], sw)

        @pl.when(g + 1 < _BATCH_PER_W)
        def _():
            start(g + 1)

        return carry

    lax.fori_loop(0, _BATCH_PER_W, body, 0)
    wait_write()


_sc_gather = pl.kernel(
    _sc_gather_body,
    out_type=jax.ShapeDtypeStruct((_BATCH, _SEQ, _VOCAB), jnp.float32),
    mesh=_sc_mesh,
    scratch_types=_SC_SCRATCH,
)


def kernel(input_ids, embed, W, b):
    w_pad = jnp.pad(W, ((0, _VPAD - _VOCAB), (0, 0)))
    b_pad = jnp.pad(b, (0, _VPAD - _VOCAB))
    t3 = _make_table3(embed, w_pad, b_pad)
    ids_pad = jnp.pad(
        input_ids.astype(jnp.int32), ((0, 0), (0, _SEQ_PAD - _SEQ))
    ).reshape(_BATCH * _SEQ_PAD)
    return _sc_gather(t3, ids_pad)
